# baseline (speedup 1.0000x reference)
import functools

import jax
import jax.numpy as jnp
from jax import lax
from jax.experimental import pallas as pl
from jax.experimental.pallas import tpu as pltpu
from jax.experimental.pallas import tpu_sc as plsc

D_MODEL = 64
SCALE = 8.0
CHUNK = 128
NBUF = 4


@functools.lru_cache(maxsize=None)
def _make_kernel(B: int):
    info = plsc.get_sparse_core_info()
    nc, ns = info.num_cores, info.num_subcores
    nw = nc * ns
    b_per_w = B // nw
    n_chunks = b_per_w // CHUNK
    n_outer = n_chunks // NBUF
    assert b_per_w * nw == B and n_outer * NBUF == n_chunks

    mesh = plsc.VectorSubcoreMesh(core_axis_name="c", subcore_axis_name="s")

    @functools.partial(
        pl.kernel,
        mesh=mesh,
        out_type=jax.ShapeDtypeStruct((B, D_MODEL), jnp.float32),
        compiler_params=pltpu.CompilerParams(use_tc_tiling_on_sc=False),
        scratch_types=(
            [pltpu.VMEM((b_per_w,), jnp.int32)]
            + [pltpu.VMEM((CHUNK, D_MODEL), jnp.float32) for _ in range(NBUF)]
            + [pltpu.SemaphoreType.DMA for _ in range(NBUF)]
        ),
    )
    def emb_kernel(x_hbm, lut_hbm, out_hbm, idx_v, *rest):
        bufs = rest[:NBUF]
        sems = rest[NBUF:]
        wid = lax.axis_index("s") * nc + lax.axis_index("c")
        base = wid * b_per_w

        pltpu.sync_copy(x_hbm.at[pl.ds(base, b_per_w)], idx_v)

        def start_gather(g, b):
            idx_slice = idx_v.at[pl.ds(pl.multiple_of(g * CHUNK, CHUNK), CHUNK)]
            pltpu.async_copy(lut_hbm.at[idx_slice], bufs[b], sems[b])

        def wait_gather(g, b):
            idx_slice = idx_v.at[pl.ds(pl.multiple_of(g * CHUNK, CHUNK), CHUNK)]
            pltpu.make_async_copy(lut_hbm.at[idx_slice], bufs[b], sems[b]).wait()

        for b in range(NBUF):
            start_gather(b, b)

        def outer(o, carry):
            g0 = o * NBUF
            for b in range(NBUF):
                g = g0 + b
                wait_gather(g, b)

                buf = bufs[b]

                def scale_rows(r, c2, buf=buf):
                    for c in range(D_MODEL // 16):
                        sl = pl.ds(c * 16, 16)
                        buf[r, sl] = buf[r, sl] * SCALE
                    return c2

                lax.fori_loop(0, CHUNK, scale_rows, 0, unroll=4)

                pltpu.sync_copy(
                    buf,
                    out_hbm.at[pl.ds(base + pl.multiple_of(g * CHUNK, CHUNK), CHUNK)],
                )

                @pl.when(g + NBUF < n_chunks)
                def _():
                    start_gather(g + NBUF, b)
            return carry

        lax.fori_loop(0, n_outer, outer, 0)

    return emb_kernel


def kernel(x, lut):
    B = x.shape[0] * x.shape[1]
    xf = x.reshape(B).astype(jnp.int32)
    out = _make_kernel(B)(xf, lut)
    return out.reshape(x.shape[0], x.shape[1], D_MODEL)
